# baseline (speedup 1.0000x reference)
import functools

import jax
import jax.numpy as jnp
from jax import lax
from jax.experimental import pallas as pl
from jax.experimental.pallas import tpu as pltpu
from jax.experimental.pallas import tpu_sc as plsc


_BN = 8192
_LANES = 16


def _tb_body(gf_ref, qt_ref, tb_ref):
    tb_ref[...] = jnp.sum(gf_ref[...] + qt_ref[...], axis=1)


def _rowsum_body(nt_ref, sv_ref, rs_ref):
    ones = jnp.ones((1, nt_ref.shape[1]), jnp.float32)
    rs_ref[...] = jnp.dot(ones, (nt_ref[...] + sv_ref[...]).T)[0]


def _combine_body(rs_ref, g_ref, out_ref):
    out_ref[...] = rs_ref[...] + g_ref[...]


def _make_sc_gather(n, np_, b, n_workers):
    chunk = np_ // n_workers
    mesh = plsc.VectorSubcoreMesh(core_axis_name="c", subcore_axis_name="s",
                                  num_cores=1)

    @functools.partial(
        pl.kernel,
        mesh=mesh,
        out_type=jax.ShapeDtypeStruct((np_,), jnp.float32),
        scratch_types=[
            pltpu.VMEM((chunk,), jnp.int32),
            pltpu.VMEM((b,), jnp.float32),
            pltpu.VMEM((chunk,), jnp.float32),
        ],
        compiler_params=pltpu.CompilerParams(needs_layout_passes=False),
    )
    def sc_gather(tb_hbm, idx_hbm, out_hbm, idx_v, tb_v, out_v):
        num_c = jax.lax.axis_size("c")
        wid = lax.axis_index("s") * num_c + lax.axis_index("c")
        base = wid * chunk
        pltpu.sync_copy(idx_hbm.at[pl.ds(base, chunk)], idx_v)
        pltpu.sync_copy(tb_hbm, tb_v)

        def body(j, carry):
            sl = pl.ds(j * _LANES, _LANES)
            ix = jnp.minimum(jnp.maximum(idx_v[sl], 0), b - 1)
            out_v[sl] = plsc.load_gather(tb_v, [ix])
            return carry

        lax.fori_loop(0, chunk // _LANES, body, 0, unroll=4)
        pltpu.sync_copy(out_v, out_hbm.at[pl.ds(base, chunk)])

    return sc_gather


def kernel(node_tokens, question_tokens, graph_features, state_vec, node_batch):
    n, d = node_tokens.shape
    b = question_tokens.shape[0]
    nb = (n + _BN - 1) // _BN
    np_ = nb * _BN
    idx = node_batch.astype(jnp.int32)

    tb = pl.pallas_call(
        _tb_body,
        in_specs=[pl.BlockSpec((b, d), lambda: (0, 0)),
                  pl.BlockSpec((b, d), lambda: (0, 0))],
        out_specs=pl.BlockSpec((b,), lambda: (0,)),
        out_shape=jax.ShapeDtypeStruct((b,), jnp.float32),
    )(graph_features, question_tokens)

    info = plsc.get_sparse_core_info()
    n_workers = info.num_subcores
    g = _make_sc_gather(n, np_, b, n_workers)(tb, idx)

    rs = pl.pallas_call(
        _rowsum_body,
        grid=(nb,),
        in_specs=[
            pl.BlockSpec((_BN, d), lambda i: (i, 0)),
            pl.BlockSpec((_BN, d), lambda i: (i, 0)),
        ],
        out_specs=pl.BlockSpec((_BN,), lambda i: (i,)),
        out_shape=jax.ShapeDtypeStruct((np_,), jnp.float32),
    )(node_tokens, state_vec)

    out = pl.pallas_call(
        _combine_body,
        grid=(nb,),
        in_specs=[
            pl.BlockSpec((_BN,), lambda i: (i,)),
            pl.BlockSpec((_BN,), lambda i: (i,)),
        ],
        out_specs=pl.BlockSpec((_BN,), lambda i: (i,)),
        out_shape=jax.ShapeDtypeStruct((n,), jnp.float32),
    )(rs, g)
    return out


# speedup vs baseline: 1.0672x; 1.0672x over previous
import functools

import jax
import jax.numpy as jnp
from jax import lax
from jax.experimental import pallas as pl
from jax.experimental.pallas import tpu as pltpu
from jax.experimental.pallas import tpu_sc as plsc


_BN = 8192
_LANES = 16


def _tb_body(gf_ref, qt_ref, tb_ref):
    tb_ref[...] = jnp.sum(gf_ref[...] + qt_ref[...], axis=1)


def _rowsum_body(nt_ref, sv_ref, rs_ref):
    ones = jnp.ones((1, nt_ref.shape[1]), jnp.float32)
    rs_ref[...] = jnp.dot(ones, (nt_ref[...] + sv_ref[...]).T)[0]


def _combine_body(rs_ref, g_ref, out_ref):
    out_ref[...] = rs_ref[...] + g_ref[...]


def _make_sc_gather(n, np_, b, n_workers):
    chunk = np_ // n_workers
    mesh = plsc.VectorSubcoreMesh(core_axis_name="c", subcore_axis_name="s",
                                  num_cores=1)

    @functools.partial(
        pl.kernel,
        mesh=mesh,
        out_type=jax.ShapeDtypeStruct((np_,), jnp.float32),
        scratch_types=[
            pltpu.VMEM((chunk,), jnp.int32),
            pltpu.VMEM((b,), jnp.float32),
            pltpu.VMEM((chunk,), jnp.float32),
        ],
        compiler_params=pltpu.CompilerParams(needs_layout_passes=False),
    )
    def sc_gather(tb_hbm, idx_hbm, out_hbm, idx_v, tb_v, out_v):
        num_c = jax.lax.axis_size("c")
        wid = lax.axis_index("s") * num_c + lax.axis_index("c")
        base = wid * chunk
        pltpu.sync_copy(idx_hbm.at[pl.ds(base, chunk)], idx_v)
        pltpu.sync_copy(tb_hbm, tb_v)

        def body(j, carry):
            sl = pl.ds(j * _LANES, _LANES)
            ix = jnp.minimum(jnp.maximum(idx_v[sl], 0), b - 1)
            out_v[sl] = plsc.load_gather(tb_v, [ix])
            return carry

        lax.fori_loop(0, chunk // _LANES, body, 0, unroll=4)
        pltpu.sync_copy(out_v, out_hbm.at[pl.ds(base, chunk)])

    return sc_gather


def kernel(node_tokens, question_tokens, graph_features, state_vec, node_batch):
    n, d = node_tokens.shape
    b = question_tokens.shape[0]
    nb = (n + _BN - 1) // _BN
    np_ = nb * _BN
    idx = node_batch.astype(jnp.int32)

    tb = pl.pallas_call(
        _tb_body,
        in_specs=[pl.BlockSpec((b, d), lambda: (0, 0)),
                  pl.BlockSpec((b, d), lambda: (0, 0))],
        out_specs=pl.BlockSpec((b,), lambda: (0,)),
        out_shape=jax.ShapeDtypeStruct((b,), jnp.float32),
    )(graph_features, question_tokens)

    info = plsc.get_sparse_core_info()
    n_workers = info.num_subcores
    g = _make_sc_gather(n, np_, b, n_workers)(tb, idx)

    rs = pl.pallas_call(
        _rowsum_body,
        grid=(nb,),
        in_specs=[
            pl.BlockSpec((_BN, d), lambda i: (i, 0)),
            pl.BlockSpec((_BN, d), lambda i: (i, 0)),
        ],
        out_specs=pl.BlockSpec((_BN,), lambda i: (i,)),
        out_shape=jax.ShapeDtypeStruct((np_,), jnp.float32),
    )(node_tokens, state_vec)

    rows = np_ // 128
    out = pl.pallas_call(
        _combine_body,
        in_specs=[pl.BlockSpec((rows, 128), lambda: (0, 0)),
                  pl.BlockSpec((rows, 128), lambda: (0, 0))],
        out_specs=pl.BlockSpec((rows, 128), lambda: (0, 0)),
        out_shape=jax.ShapeDtypeStruct((rows, 128), jnp.float32),
    )(rs.reshape(rows, 128), g.reshape(rows, 128))
    return out.reshape(np_)[:n]
